# Initial kernel scaffold; baseline (speedup 1.0000x reference)
import jax
import jax.numpy as jnp
from jax.experimental import pallas as pl


def kernel(emb, W1, R1, B1, W2, R2, B2, W3, R3, B3, W4, R4, B4, g1, be1, g2, be2, ei, et):
    raise NotImplementedError("write your pallas kernel here")


# speedup vs baseline: 10.5620x; 10.5620x over previous
import functools

import jax
import jax.numpy as jnp
from jax import lax
from jax.experimental import pallas as pl
from jax.experimental.pallas import tpu as pltpu
from jax.experimental.pallas import tpu_sc as plsc

N = 10000
R = 8
E = 640000
NC = 2
NS = 16
NW = NC * NS
EPW = E // NW
KB = 80
NBAT = EPW // KB
CH = 800
NCH = EPW // CH
NP = 10240
RPT = NP // NS
ZROWS = 64
BPC = CH // KB

_MESH = dict(core_axis_name="c", subcore_axis_name="s", num_cores=NC,
             num_subcores=NS)


def _counts_body(dst1, et1, out, dch, ech, drow, onehot, zbuf, acc):
  cid = lax.axis_index("c")
  sid = lax.axis_index("s")
  wid = sid * NC + cid

  zero16 = jnp.zeros((16,), jnp.float32)
  ones = jnp.ones((16,), jnp.float32)
  zeros = jnp.zeros((16,), jnp.float32)
  lanes = lax.iota(jnp.int32, 16)

  for i in range(KB):
    for f in range(8):
      onehot[i, pl.ds(16 * f, 16)] = zero16
  for i in range(ZROWS):
    for f in range(8):
      zbuf[i, pl.ds(16 * f, 16)] = zero16
  for c in range(RPT // ZROWS):
    pltpu.sync_copy(zbuf, acc.at[pl.ds(sid * RPT + c * ZROWS, ZROWS)])
  plsc.subcore_barrier()

  def chunk(cc, carry0):
    base = wid * EPW + cc * CH
    pltpu.sync_copy(dst1.at[pl.ds(base, CH)], dch)
    pltpu.sync_copy(et1.at[pl.ds(base, CH)], ech)

    def batch(b, c):
      for j in range(KB // 16):
        drow[pl.ds(j * 16, 16)] = dch[pl.ds(b * KB + j * 16, 16)]
        ev16 = ech[pl.ds(b * KB + j * 16, 16)]
        plsc.store_scatter(onehot, [lanes + j * 16, ev16], ones)
      pltpu.sync_copy(onehot, acc.at[drow], add=True)
      for j in range(KB // 16):
        ev16 = ech[pl.ds(b * KB + j * 16, 16)]
        plsc.store_scatter(onehot, [lanes + j * 16, ev16], zeros)
      return c
    lax.fori_loop(0, BPC, batch, 0)
    return carry0
  lax.fori_loop(0, NCH, chunk, 0)
  plsc.subcore_barrier()
  for c in range(RPT // ZROWS):
    base = sid * RPT + c * ZROWS
    pltpu.sync_copy(acc.at[pl.ds(base, ZROWS)], out.at[cid, pl.ds(base, ZROWS)])


_counts_kernel = pl.kernel(
    _counts_body,
    out_type=jax.ShapeDtypeStruct((NC, NP, 128), jnp.float32),
    mesh=plsc.VectorSubcoreMesh(**_MESH),
    compiler_params=pltpu.CompilerParams(needs_layout_passes=False),
    scratch_types=[
        pltpu.VMEM((CH,), jnp.int32),
        pltpu.VMEM((CH,), jnp.int32),
        pltpu.VMEM((KB,), jnp.int32),
        pltpu.VMEM((KB, 128), jnp.float32),
        pltpu.VMEM((ZROWS, 128), jnp.float32),
        pltpu.VMEM_SHARED((NP, 128), jnp.float32),
    ],
)


def _scale_body(invf, src1, et1, dst1, qout, sout, inv_v, srcb, etb, dstb, qb,
                sb):
  cid = lax.axis_index("c")
  sid = lax.axis_index("s")
  wid = sid * NC + cid
  lanes = lax.iota(jnp.int32, 16)
  pltpu.sync_copy(invf, inv_v)

  def chunk(c, carry):
    base = wid * EPW + c * CH
    pltpu.sync_copy(src1.at[pl.ds(base, CH)], srcb)
    pltpu.sync_copy(et1.at[pl.ds(base, CH)], etb)
    pltpu.sync_copy(dst1.at[pl.ds(base, CH)], dstb)

    def step(j, c2):
      off = j * 16
      sv = srcb[pl.ds(off, 16)]
      ev = etb[pl.ds(off, 16)]
      dv = dstb[pl.ds(off, 16)]
      idx = lanes + off
      plsc.store_scatter(qb, [idx], sv * 9 + ev)
      plsc.store_scatter(sb, [idx], plsc.load_gather(inv_v, [dv * R + ev]))
      return c2
    lax.fori_loop(0, CH // 16, step, 0)
    pltpu.sync_copy(qb, qout.at[pl.ds(base, CH)])
    pltpu.sync_copy(sb, sout.at[pl.ds(base, CH)])
    return carry
  lax.fori_loop(0, NCH, chunk, 0)


_scale_kernel = pl.kernel(
    _scale_body,
    out_type=(jax.ShapeDtypeStruct((E,), jnp.int32),
              jax.ShapeDtypeStruct((E,), jnp.float32)),
    mesh=plsc.VectorSubcoreMesh(**_MESH),
    compiler_params=pltpu.CompilerParams(needs_layout_passes=False),
    scratch_types=[
        pltpu.VMEM((N * R,), jnp.float32),
        pltpu.VMEM((CH,), jnp.int32),
        pltpu.VMEM((CH,), jnp.int32),
        pltpu.VMEM((CH,), jnp.int32),
        pltpu.VMEM((CH,), jnp.int32),
        pltpu.VMEM((CH,), jnp.float32),
    ],
)


def _make_agg_kernel(D):
  FB = D // 16

  def body(ytab, q1, s1, dst1, out, qch, sch, dch, qrow, drow, rows, zbuf,
           sem, acc):
    cid = lax.axis_index("c")
    sid = lax.axis_index("s")
    wid = sid * NC + cid

    zero16 = jnp.zeros((16,), jnp.float32)
    lanes = lax.iota(jnp.int32, 16)
    for i in range(ZROWS):
      for f in range(FB):
        zbuf[i, pl.ds(16 * f, 16)] = zero16

    for c in range(RPT // ZROWS):
      pltpu.sync_copy(zbuf, acc.at[pl.ds(sid * RPT + c * ZROWS, ZROWS)])
    plsc.subcore_barrier()

    def chunk(cc, carry0):
      base = wid * EPW + cc * CH
      pltpu.sync_copy(q1.at[pl.ds(base, CH)], qch)
      pltpu.sync_copy(s1.at[pl.ds(base, CH)], sch)
      pltpu.sync_copy(dst1.at[pl.ds(base, CH)], dch)

      def batch(b, carry):
        for j in range(KB // 16):
          qrow[pl.ds(16 * j, 16)] = qch[pl.ds(b * KB + 16 * j, 16)]
          drow[pl.ds(16 * j, 16)] = dch[pl.ds(b * KB + 16 * j, 16)]
        pltpu.async_copy(ytab.at[qrow], rows, sem).wait()

        def scale_edge(k, c2):
          sc = plsc.load_gather(sch, [jnp.full((16,), b * KB + k, jnp.int32)])
          kv = jnp.full((16,), k, jnp.int32)
          for f in range(FB):
            cv = lanes + 16 * f
            v = plsc.load_gather(rows, [kv, cv])
            plsc.store_scatter(rows, [kv, cv], v * sc)
          return c2
        lax.fori_loop(0, KB, scale_edge, 0)
        pltpu.sync_copy(rows, acc.at[drow], add=True)
        return carry
      lax.fori_loop(0, BPC, batch, 0)
      return carry0
    lax.fori_loop(0, NCH, chunk, 0)
    plsc.subcore_barrier()
    for c in range(RPT // ZROWS):
      base = sid * RPT + c * ZROWS
      pltpu.sync_copy(acc.at[pl.ds(base, ZROWS)],
                      out.at[cid, pl.ds(base, ZROWS)])

  return pl.kernel(
      body,
      out_type=jax.ShapeDtypeStruct((NC, NP, D), jnp.float32),
      mesh=plsc.VectorSubcoreMesh(**_MESH),
      compiler_params=pltpu.CompilerParams(needs_layout_passes=False),
      scratch_types=[
          pltpu.VMEM((CH,), jnp.int32),
          pltpu.VMEM((CH,), jnp.float32),
          pltpu.VMEM((CH,), jnp.int32),
          pltpu.VMEM((KB,), jnp.int32),
          pltpu.VMEM((KB,), jnp.int32),
          pltpu.VMEM((KB, D), jnp.float32),
          pltpu.VMEM((ZROWS, D), jnp.float32),
          pltpu.SemaphoreType.DMA,
          pltpu.VMEM_SHARED((NP, D), jnp.float32),
      ],
  )


_agg128 = _make_agg_kernel(128)


BN = 1000
GRID = N // BN
W9 = 9 * 128


def _ln(x, g, b):
  m = jnp.mean(x, axis=-1, keepdims=True)
  v = jnp.mean((x - m) ** 2, axis=-1, keepdims=True)
  return (x - m) / jnp.sqrt(v + 1e-5) * g + b


def _row_spec(d):
  return pl.BlockSpec((BN, d), lambda i: (i, 0))


def _full_spec(shape):
  return pl.BlockSpec(shape, lambda i: tuple(0 for _ in shape))


def _m1_body(emb, w1cat, cnt0, cnt1, ycat, inv):
  ycat[...] = jnp.dot(emb[...], w1cat[...],
                      preferred_element_type=jnp.float32)
  cnt = cnt0[...] + cnt1[...]
  inv[...] = 1.0 / jnp.maximum(cnt[:, :R], 1.0)


def _m1(emb, w1cat, cnt0, cnt1):
  return pl.pallas_call(
      _m1_body,
      grid=(GRID,),
      in_specs=[_row_spec(128), _full_spec((128, W9)),
                _row_spec(128), _row_spec(128)],
      out_specs=[_row_spec(W9), _row_spec(R)],
      out_shape=[jax.ShapeDtypeStruct((N, W9), jnp.float32),
                 jax.ShapeDtypeStruct((N, R), jnp.float32)],
  )(emb, w1cat, cnt0, cnt1)


def _m2_body(ycat, b1, p0, p1, g1, be1, w2cat, x1, y2cat):
  c = ycat[:, 8 * 128:] + b1[...] + p0[...] + p1[...]
  x = _ln(jax.nn.relu(c), g1[...], be1[...])
  x1[...] = x
  y2cat[...] = jnp.dot(x, w2cat[...], preferred_element_type=jnp.float32)


def _m2(ycat, b1, p0, p1, g1, be1, w2cat):
  return pl.pallas_call(
      _m2_body,
      grid=(GRID,),
      in_specs=[_row_spec(W9), _full_spec((1, 128)), _row_spec(128),
                _row_spec(128), _full_spec((1, 128)), _full_spec((1, 128)),
                _full_spec((128, W9))],
      out_specs=[_row_spec(128), _row_spec(W9)],
      out_shape=[jax.ShapeDtypeStruct((N, 128), jnp.float32),
                 jax.ShapeDtypeStruct((N, W9), jnp.float32)],
  )(ycat, b1, p0, p1, g1, be1, w2cat)


def _m3_body(ycat, b2, p0, p1, ident, g2, be2, w3cat, y3cat):
  c = ycat[:, 8 * 128:] + b2[...] + p0[...] + p1[...]
  x = _ln(jax.nn.relu(c) + ident[...], g2[...], be2[...])
  y3cat[...] = jnp.dot(x, w3cat[...], preferred_element_type=jnp.float32)


def _m3(ycat, b2, p0, p1, ident, g2, be2, w3cat):
  return pl.pallas_call(
      _m3_body,
      grid=(GRID,),
      in_specs=[_row_spec(W9), _full_spec((1, 128)), _row_spec(128),
                _row_spec(128), _row_spec(128), _full_spec((1, 128)),
                _full_spec((1, 128)), _full_spec((128, W9))],
      out_specs=[_row_spec(W9)],
      out_shape=[jax.ShapeDtypeStruct((N, W9), jnp.float32)],
  )(ycat, b2, p0, p1, ident, g2, be2, w3cat)[0]


def _m4_body(y3cat, b3, p0, p1, w4cat, y4cat):
  c = (y3cat[:, 8 * 128:8 * 128 + 64] + b3[...] + p0[:, :64] + p1[:, :64])
  x = jax.nn.relu(c)
  y4cat[...] = jnp.dot(x, w4cat[...], preferred_element_type=jnp.float32)


def _m4(y3cat, b3, p0, p1, w4cat):
  return pl.pallas_call(
      _m4_body,
      grid=(GRID,),
      in_specs=[_row_spec(W9), _full_spec((1, 64)), _row_spec(128),
                _row_spec(128), _full_spec((64, W9))],
      out_specs=[_row_spec(W9)],
      out_shape=[jax.ShapeDtypeStruct((N, W9), jnp.float32)],
  )(y3cat, b3, p0, p1, w4cat)[0]


def _m5_body(y4cat, b4, p0, p1, out):
  out[...] = (y4cat[:, 8 * 128:8 * 128 + 64] + b4[...] + p0[:, :64]
              + p1[:, :64])


def _m5(y4cat, b4, p0, p1):
  return pl.pallas_call(
      _m5_body,
      grid=(GRID,),
      in_specs=[_row_spec(W9), _full_spec((1, 64)), _row_spec(128),
                _row_spec(128)],
      out_specs=[_row_spec(64)],
      out_shape=[jax.ShapeDtypeStruct((N, 64), jnp.float32)],
  )(y4cat, b4, p0, p1)[0]


def _wcat(W, root):
  din, dout = W.shape[1], W.shape[2]
  Wt = jnp.transpose(W, (1, 0, 2))
  if dout < 128:
    Wt = jnp.pad(Wt, ((0, 0), (0, 0), (0, 128 - dout)))
    rootp = jnp.pad(root, ((0, 0), (0, 128 - dout)))
  else:
    rootp = root
  return jnp.concatenate([Wt.reshape(din, R * 128), rootp], axis=1)


def kernel(emb, W1, R1, B1, W2, R2, B2, W3, R3, B3, W4, R4, B4, g1, be1, g2,
           be2, ei, et):
  src = ei[0]
  dst = ei[1]

  cntp = _counts_kernel(dst, et)

  ycat1, inv = _m1(emb, _wcat(W1, R1), cntp[0, :N], cntp[1, :N])

  q1, s1 = _scale_kernel(inv.reshape(N * R), src, et, dst)

  p1 = _agg128(ycat1.reshape(N * 9, 128), q1, s1, dst)
  x1, ycat2 = _m2(ycat1, B1.reshape(1, 128), p1[0, :N], p1[1, :N],
                  g1.reshape(1, 128), be1.reshape(1, 128), _wcat(W2, R2))

  p2 = _agg128(ycat2.reshape(N * 9, 128), q1, s1, dst)
  ycat3 = _m3(ycat2, B2.reshape(1, 128), p2[0, :N], p2[1, :N], x1,
              g2.reshape(1, 128), be2.reshape(1, 128), _wcat(W3, R3))

  p3 = _agg128(ycat3.reshape(N * 9, 128), q1, s1, dst)
  ycat4 = _m4(ycat3, B3.reshape(1, 64), p3[0, :N], p3[1, :N], _wcat(W4, R4))

  p4 = _agg128(ycat4.reshape(N * 9, 128), q1, s1, dst)
  return _m5(ycat4, B4.reshape(1, 64), p4[0, :N], p4[1, :N])


# speedup vs baseline: 10.5735x; 1.0011x over previous
import jax
import jax.numpy as jnp
from jax import lax
from jax.experimental import pallas as pl
from jax.experimental.pallas import tpu as pltpu
from jax.experimental.pallas import tpu_sc as plsc

N = 10000
R = 8
E = 640000
NC = 2
NS = 16
NW = NC * NS
EPW = E // NW
KB = 80
NBAT = EPW // KB
CH = 800
NCH = EPW // CH
NP = 10240
RPT = NP // NS
ZROWS = 64
BPC = CH // KB

_MESH = dict(core_axis_name="c", subcore_axis_name="s", num_cores=NC,
             num_subcores=NS)


def _counts_body(dst1, et1, out, dch, ech, drow, onehot, zbuf, acc):
  cid = lax.axis_index("c")
  sid = lax.axis_index("s")
  wid = sid * NC + cid

  zero16 = jnp.zeros((16,), jnp.float32)
  ones = jnp.ones((16,), jnp.float32)
  zeros = jnp.zeros((16,), jnp.float32)
  lanes = lax.iota(jnp.int32, 16)

  for i in range(KB):
    for f in range(8):
      onehot[i, pl.ds(16 * f, 16)] = zero16
  for i in range(ZROWS):
    for f in range(8):
      zbuf[i, pl.ds(16 * f, 16)] = zero16
  for c in range(RPT // ZROWS):
    pltpu.sync_copy(zbuf, acc.at[pl.ds(sid * RPT + c * ZROWS, ZROWS)])
  plsc.subcore_barrier()

  def chunk(cc, carry0):
    base = wid * EPW + cc * CH
    pltpu.sync_copy(dst1.at[pl.ds(base, CH)], dch)
    pltpu.sync_copy(et1.at[pl.ds(base, CH)], ech)

    def batch(b, c):
      for j in range(KB // 16):
        drow[pl.ds(j * 16, 16)] = dch[pl.ds(b * KB + j * 16, 16)]
        ev16 = ech[pl.ds(b * KB + j * 16, 16)]
        plsc.store_scatter(onehot, [lanes + j * 16, ev16], ones)
      pltpu.sync_copy(onehot, acc.at[drow], add=True)
      for j in range(KB // 16):
        ev16 = ech[pl.ds(b * KB + j * 16, 16)]
        plsc.store_scatter(onehot, [lanes + j * 16, ev16], zeros)
      return c
    lax.fori_loop(0, BPC, batch, 0)
    return carry0
  lax.fori_loop(0, NCH, chunk, 0)
  plsc.subcore_barrier()
  for c in range(RPT // ZROWS):
    base = sid * RPT + c * ZROWS
    pltpu.sync_copy(acc.at[pl.ds(base, ZROWS)], out.at[cid, pl.ds(base, ZROWS)])


_counts_kernel = pl.kernel(
    _counts_body,
    out_type=jax.ShapeDtypeStruct((NC, NP, 128), jnp.float32),
    mesh=plsc.VectorSubcoreMesh(**_MESH),
    compiler_params=pltpu.CompilerParams(needs_layout_passes=False),
    scratch_types=[
        pltpu.VMEM((CH,), jnp.int32),
        pltpu.VMEM((CH,), jnp.int32),
        pltpu.VMEM((KB,), jnp.int32),
        pltpu.VMEM((KB, 128), jnp.float32),
        pltpu.VMEM((ZROWS, 128), jnp.float32),
        pltpu.VMEM_SHARED((NP, 128), jnp.float32),
    ],
)


def _scale_body(invf, src1, et1, dst1, qout, sout, inv_v, srcb, etb, dstb, qb,
                sb):
  cid = lax.axis_index("c")
  sid = lax.axis_index("s")
  wid = sid * NC + cid
  lanes = lax.iota(jnp.int32, 16)
  pltpu.sync_copy(invf, inv_v)

  def chunk(c, carry):
    base = wid * EPW + c * CH
    pltpu.sync_copy(src1.at[pl.ds(base, CH)], srcb)
    pltpu.sync_copy(et1.at[pl.ds(base, CH)], etb)
    pltpu.sync_copy(dst1.at[pl.ds(base, CH)], dstb)

    def step(j, c2):
      off = j * 16
      sv = srcb[pl.ds(off, 16)]
      ev = etb[pl.ds(off, 16)]
      dv = dstb[pl.ds(off, 16)]
      idx = lanes + off
      plsc.store_scatter(qb, [idx], sv * 9 + ev)
      plsc.store_scatter(sb, [idx], plsc.load_gather(inv_v, [dv * R + ev]))
      return c2
    lax.fori_loop(0, CH // 16, step, 0)
    pltpu.sync_copy(qb, qout.at[pl.ds(base, CH)])
    pltpu.sync_copy(sb, sout.at[pl.ds(base, CH)])
    return carry
  lax.fori_loop(0, NCH, chunk, 0)


_scale_kernel = pl.kernel(
    _scale_body,
    out_type=(jax.ShapeDtypeStruct((E,), jnp.int32),
              jax.ShapeDtypeStruct((E,), jnp.float32)),
    mesh=plsc.VectorSubcoreMesh(**_MESH),
    compiler_params=pltpu.CompilerParams(needs_layout_passes=False),
    scratch_types=[
        pltpu.VMEM((N * R,), jnp.float32),
        pltpu.VMEM((CH,), jnp.int32),
        pltpu.VMEM((CH,), jnp.int32),
        pltpu.VMEM((CH,), jnp.int32),
        pltpu.VMEM((CH,), jnp.int32),
        pltpu.VMEM((CH,), jnp.float32),
    ],
)


def _make_agg_kernel(D):
  FB = D // 16

  def body(ytab, q1, s1, dst1, out, qch, sch, dch, qrow, drow, rows, zbuf,
           sem, acc):
    cid = lax.axis_index("c")
    sid = lax.axis_index("s")
    wid = sid * NC + cid

    zero16 = jnp.zeros((16,), jnp.float32)
    lanes = lax.iota(jnp.int32, 16)
    for i in range(ZROWS):
      for f in range(FB):
        zbuf[i, pl.ds(16 * f, 16)] = zero16

    for c in range(RPT // ZROWS):
      pltpu.sync_copy(zbuf, acc.at[pl.ds(sid * RPT + c * ZROWS, ZROWS)])
    plsc.subcore_barrier()

    def chunk(cc, carry0):
      base = wid * EPW + cc * CH
      pltpu.sync_copy(q1.at[pl.ds(base, CH)], qch)
      pltpu.sync_copy(s1.at[pl.ds(base, CH)], sch)
      pltpu.sync_copy(dst1.at[pl.ds(base, CH)], dch)

      def batch(b, carry):
        for j in range(KB // 16):
          qrow[pl.ds(16 * j, 16)] = qch[pl.ds(b * KB + 16 * j, 16)]
          drow[pl.ds(16 * j, 16)] = dch[pl.ds(b * KB + 16 * j, 16)]
        pltpu.async_copy(ytab.at[qrow], rows, sem).wait()

        def scale_edge(k, c2):
          sc = plsc.load_gather(sch, [jnp.full((16,), b * KB + k, jnp.int32)])
          kv = jnp.full((16,), k, jnp.int32)
          for f in range(FB):
            cv = lanes + 16 * f
            v = plsc.load_gather(rows, [kv, cv])
            plsc.store_scatter(rows, [kv, cv], v * sc)
          return c2
        lax.fori_loop(0, KB, scale_edge, 0)
        pltpu.sync_copy(rows, acc.at[drow], add=True)
        return carry
      lax.fori_loop(0, BPC, batch, 0)
      return carry0
    lax.fori_loop(0, NCH, chunk, 0)
    plsc.subcore_barrier()
    for c in range(RPT // ZROWS):
      base = sid * RPT + c * ZROWS
      pltpu.sync_copy(acc.at[pl.ds(base, ZROWS)],
                      out.at[cid, pl.ds(base, ZROWS)])

  return pl.kernel(
      body,
      out_type=jax.ShapeDtypeStruct((NC, NP, D), jnp.float32),
      mesh=plsc.VectorSubcoreMesh(**_MESH),
      compiler_params=pltpu.CompilerParams(needs_layout_passes=False),
      scratch_types=[
          pltpu.VMEM((CH,), jnp.int32),
          pltpu.VMEM((CH,), jnp.float32),
          pltpu.VMEM((CH,), jnp.int32),
          pltpu.VMEM((KB,), jnp.int32),
          pltpu.VMEM((KB,), jnp.int32),
          pltpu.VMEM((KB, D), jnp.float32),
          pltpu.VMEM((ZROWS, D), jnp.float32),
          pltpu.SemaphoreType.DMA,
          pltpu.VMEM_SHARED((NP, D), jnp.float32),
      ],
  )


_agg128 = _make_agg_kernel(128)


BN = 1000
GRID = N // BN
W9 = 9 * 128


def _ln(x, g, b):
  m = jnp.mean(x, axis=-1, keepdims=True)
  v = jnp.mean((x - m) ** 2, axis=-1, keepdims=True)
  return (x - m) / jnp.sqrt(v + 1e-5) * g + b


def _row_spec(d):
  return pl.BlockSpec((BN, d), lambda i: (i, 0))


def _full_spec(shape):
  return pl.BlockSpec(shape, lambda i: tuple(0 for _ in shape))


def _m1_body(emb, w1cat, cnt0, cnt1, ycat, inv):
  ycat[...] = jnp.dot(emb[...], w1cat[...],
                      preferred_element_type=jnp.float32)
  cnt = cnt0[...] + cnt1[...]
  inv[...] = 1.0 / jnp.maximum(cnt[:, :R], 1.0)


def _m1(emb, w1cat, cnt0, cnt1):
  return pl.pallas_call(
      _m1_body,
      grid=(GRID,),
      in_specs=[_row_spec(128), _full_spec((128, W9)),
                _row_spec(128), _row_spec(128)],
      out_specs=[_row_spec(W9), _row_spec(R)],
      out_shape=[jax.ShapeDtypeStruct((N, W9), jnp.float32),
                 jax.ShapeDtypeStruct((N, R), jnp.float32)],
  )(emb, w1cat, cnt0, cnt1)


def _m2_body(ycat, b1, p0, p1, g1, be1, w2cat, x1, y2cat):
  c = ycat[:, 8 * 128:] + b1[...] + p0[...] + p1[...]
  x = _ln(jax.nn.relu(c), g1[...], be1[...])
  x1[...] = x
  y2cat[...] = jnp.dot(x, w2cat[...], preferred_element_type=jnp.float32)


def _m2(ycat, b1, p0, p1, g1, be1, w2cat):
  return pl.pallas_call(
      _m2_body,
      grid=(GRID,),
      in_specs=[_row_spec(W9), _full_spec((1, 128)), _row_spec(128),
                _row_spec(128), _full_spec((1, 128)), _full_spec((1, 128)),
                _full_spec((128, W9))],
      out_specs=[_row_spec(128), _row_spec(W9)],
      out_shape=[jax.ShapeDtypeStruct((N, 128), jnp.float32),
                 jax.ShapeDtypeStruct((N, W9), jnp.float32)],
  )(ycat, b1, p0, p1, g1, be1, w2cat)


def _m3_body(ycat, b2, p0, p1, ident, g2, be2, w3cat, y3cat):
  c = ycat[:, 8 * 128:] + b2[...] + p0[...] + p1[...]
  x = _ln(jax.nn.relu(c) + ident[...], g2[...], be2[...])
  y3cat[...] = jnp.dot(x, w3cat[...], preferred_element_type=jnp.float32)


def _m3(ycat, b2, p0, p1, ident, g2, be2, w3cat):
  return pl.pallas_call(
      _m3_body,
      grid=(GRID,),
      in_specs=[_row_spec(W9), _full_spec((1, 128)), _row_spec(128),
                _row_spec(128), _row_spec(128), _full_spec((1, 128)),
                _full_spec((1, 128)), _full_spec((128, W9))],
      out_specs=[_row_spec(W9)],
      out_shape=[jax.ShapeDtypeStruct((N, W9), jnp.float32)],
  )(ycat, b2, p0, p1, ident, g2, be2, w3cat)[0]


def _m4_body(y3cat, b3, p0, p1, w4cat, y4cat):
  c = (y3cat[:, 8 * 128:8 * 128 + 64] + b3[...] + p0[:, :64] + p1[:, :64])
  x = jax.nn.relu(c)
  y4cat[...] = jnp.dot(x, w4cat[...], preferred_element_type=jnp.float32)


def _m4(y3cat, b3, p0, p1, w4cat):
  return pl.pallas_call(
      _m4_body,
      grid=(GRID,),
      in_specs=[_row_spec(W9), _full_spec((1, 64)), _row_spec(128),
                _row_spec(128), _full_spec((64, W9))],
      out_specs=[_row_spec(W9)],
      out_shape=[jax.ShapeDtypeStruct((N, W9), jnp.float32)],
  )(y3cat, b3, p0, p1, w4cat)[0]


def _m5_body(y4cat, b4, p0, p1, out):
  out[...] = (y4cat[:, 8 * 128:8 * 128 + 64] + b4[...] + p0[:, :64]
              + p1[:, :64])


def _m5(y4cat, b4, p0, p1):
  return pl.pallas_call(
      _m5_body,
      grid=(GRID,),
      in_specs=[_row_spec(W9), _full_spec((1, 64)), _row_spec(128),
                _row_spec(128)],
      out_specs=[_row_spec(64)],
      out_shape=[jax.ShapeDtypeStruct((N, 64), jnp.float32)],
  )(y4cat, b4, p0, p1)[0]


def _wcat(W, root):
  din, dout = W.shape[1], W.shape[2]
  Wt = jnp.transpose(W, (1, 0, 2))
  if dout < 128:
    Wt = jnp.pad(Wt, ((0, 0), (0, 0), (0, 128 - dout)))
    rootp = jnp.pad(root, ((0, 0), (0, 128 - dout)))
  else:
    rootp = root
  return jnp.concatenate([Wt.reshape(din, R * 128), rootp], axis=1)


def kernel(emb, W1, R1, B1, W2, R2, B2, W3, R3, B3, W4, R4, B4, g1, be1, g2,
           be2, ei, et):
  src = ei[0]
  dst = ei[1]

  cntp = _counts_kernel(dst, et)

  ycat1, inv = _m1(emb, _wcat(W1, R1), cntp[0, :N], cntp[1, :N])

  q1, s1 = _scale_kernel(inv.reshape(N * R), src, et, dst)

  p1 = _agg128(ycat1.reshape(N * 9, 128), q1, s1, dst)
  x1, ycat2 = _m2(ycat1, B1.reshape(1, 128), p1[0, :N], p1[1, :N],
                  g1.reshape(1, 128), be1.reshape(1, 128), _wcat(W2, R2))

  p2 = _agg128(ycat2.reshape(N * 9, 128), q1, s1, dst)
  ycat3 = _m3(ycat2, B2.reshape(1, 128), p2[0, :N], p2[1, :N], x1,
              g2.reshape(1, 128), be2.reshape(1, 128), _wcat(W3, R3))

  p3 = _agg128(ycat3.reshape(N * 9, 128), q1, s1, dst)
  ycat4 = _m4(ycat3, B3.reshape(1, 64), p3[0, :N], p3[1, :N], _wcat(W4, R4))

  p4 = _agg128(ycat4.reshape(N * 9, 128), q1, s1, dst)
  return _m5(ycat4, B4.reshape(1, 64), p4[0, :N], p4[1, :N])
